# baseline (speedup 1.0000x reference)
import functools

import jax
import jax.numpy as jnp
from jax import lax
from jax.experimental import pallas as pl
from jax.experimental.pallas import tpu as pltpu
from jax.experimental.pallas import tpu_sc as plsc

N_NODES = 10000
N_EDGES = 320000
IN_DIM = 128
HIDDEN = 256

NP = 10240
NC = 2
NS = 16
NW = NC * NS
CHUNK = 128
KJ = 80
EPT = KJ * CHUNK
EPAD = NW * EPT
GCHUNK = 64
NBUF = 4
KJS = 16
UEDGES = KJS * GCHUNK
NUNITS = EPAD // UEDGES
MF = 20
MS = NUNITS // NS - MF
FAST_CORE = 1
ROWS_T = NP // NS
HALF = 128

_MESH = plsc.VectorSubcoreMesh(core_axis_name="c", subcore_axis_name="s",
                               num_cores=NC, num_subcores=NS)


def _sc_deg_body(dst3, zeros128, ones128, degp, dacc, didx, ones_v):
    c = lax.axis_index("c")
    s = lax.axis_index("s")
    wid = c * NS + s
    pltpu.sync_copy(zeros128.at[pl.ds(s * ROWS_T, ROWS_T)],
                    dacc.at[pl.ds(s * ROWS_T, ROWS_T)])
    pltpu.sync_copy(ones128, ones_v)
    pltpu.sync_copy(dst3.at[wid], didx)
    plsc.subcore_barrier()

    def body(j, carry):
        pltpu.sync_copy(ones_v, dacc.at[didx.at[j]], add=True)
        return carry

    lax.fori_loop(0, KJ, body, 0)
    plsc.subcore_barrier()
    pltpu.sync_copy(dacc.at[pl.ds(s * ROWS_T, ROWS_T)],
                    degp.at[c, pl.ds(s * ROWS_T, ROWS_T)])


def _sc_agg_body(y0, y1, src4, dst4, zeros128, aggp, acc, sidx, didx, bufs,
                 sems):
    c = lax.axis_index("c")
    s = lax.axis_index("s")
    base_u = jnp.where(c == FAST_CORE, s * MF, NS * MF + s * MS)

    for half in range(2):
        ysrc = y0 if half == 0 else y1
        pltpu.sync_copy(zeros128.at[pl.ds(s * ROWS_T, ROWS_T)],
                        acc.at[pl.ds(s * ROWS_T, ROWS_T)])
        plsc.subcore_barrier()

        def step(j, b, issue):
            pltpu.make_async_copy(ysrc.at[sidx.at[j]], bufs.at[b],
                                  sems.at[b]).wait()
            if issue:
                nb = (b + NBUF - 1) % NBUF
                pltpu.async_copy(ysrc.at[sidx.at[j + NBUF - 1]],
                                 bufs.at[nb], sems.at[nb])
            pltpu.sync_copy(bufs.at[b], acc.at[didx.at[j]], add=True)

        def do_unit(u):
            pltpu.sync_copy(src4.at[u], sidx)
            pltpu.sync_copy(dst4.at[u], didx)

            for b in range(NBUF - 1):
                pltpu.async_copy(ysrc.at[sidx.at[b]], bufs.at[b], sems.at[b])

            def body(i, carry):
                for b in range(NBUF):
                    step(NBUF * i + b, b, True)
                return carry

            lax.fori_loop(0, KJS // NBUF - 1, body, 0)
            base = KJS - NBUF
            step(base, base % NBUF, True)
            for j in range(base + 1, KJS):
                step(j, j % NBUF, False)

        for k in range(MF):
            if k < MS:
                do_unit(base_u + k)
            else:
                pl.when(c == FAST_CORE)(lambda k=k: do_unit(base_u + k))

        plsc.subcore_barrier()
        pltpu.sync_copy(
            acc.at[pl.ds(s * ROWS_T, ROWS_T)],
            aggp.at[c, pl.ds(s * ROWS_T, ROWS_T), pl.ds(half * HALF, HALF)])


_DEG_SCRATCH = [
    pltpu.VMEM_SHARED((NP, HALF), jnp.float32),
    pltpu.VMEM((KJ, CHUNK), jnp.int32),
    pltpu.VMEM((CHUNK, HALF), jnp.float32),
]
_AGG_SCRATCH = [
    pltpu.VMEM_SHARED((NP, HALF), jnp.float32),
    pltpu.VMEM((KJS, GCHUNK), jnp.int32),
    pltpu.VMEM((KJS, GCHUNK), jnp.int32),
    pltpu.VMEM((NBUF, GCHUNK, HALF), jnp.float32),
    pltpu.SemaphoreType.DMA((NBUF,)),
]
assert MF >= MS and NS * (MF + MS) == NUNITS

_sc_deg = pl.kernel(
    _sc_deg_body,
    out_type=jax.ShapeDtypeStruct((NC, NP, HALF), jnp.float32),
    mesh=_MESH,
    scratch_types=_DEG_SCRATCH,
)

_sc_agg = pl.kernel(
    _sc_agg_body,
    out_type=jax.ShapeDtypeStruct((NC, NP, HIDDEN), jnp.float32),
    mesh=_MESH,
    scratch_types=_AGG_SCRATCH,
)


_BLK = 512
_GRID1 = NP // _BLK


def _tc_layer1_body(x_ref, w_ref, b_ref, d0_ref, d1_ref,
                    y0_ref, y1_ref, z_ref, dinv_ref):
    xw = jnp.dot(x_ref[...], w_ref[...], preferred_element_type=jnp.float32)
    dinv = lax.rsqrt(d0_ref[...] + d1_ref[...] + 1.0)
    y = xw * dinv
    y0_ref[...] = y[:, :HALF]
    y1_ref[...] = y[:, HALF:]
    z_ref[...] = y * dinv + b_ref[...]
    dinv_ref[...] = dinv


def _tc_layer1(xp, W1, b1r, d0, d1):
    return pl.pallas_call(
        _tc_layer1_body,
        grid=(_GRID1,),
        in_specs=[
            pl.BlockSpec((_BLK, IN_DIM), lambda i: (i, 0)),
            pl.BlockSpec((IN_DIM, HIDDEN), lambda i: (0, 0)),
            pl.BlockSpec((1, HIDDEN), lambda i: (0, 0)),
            pl.BlockSpec((_BLK, 1), lambda i: (i, 0)),
            pl.BlockSpec((_BLK, 1), lambda i: (i, 0)),
        ],
        out_specs=[
            pl.BlockSpec((_BLK, HALF), lambda i: (i, 0)),
            pl.BlockSpec((_BLK, HALF), lambda i: (i, 0)),
            pl.BlockSpec((_BLK, HIDDEN), lambda i: (i, 0)),
            pl.BlockSpec((_BLK, 1), lambda i: (i, 0)),
        ],
        out_shape=[
            jax.ShapeDtypeStruct((NP, HALF), jnp.float32),
            jax.ShapeDtypeStruct((NP, HALF), jnp.float32),
            jax.ShapeDtypeStruct((NP, HIDDEN), jnp.float32),
            jax.ShapeDtypeStruct((NP, 1), jnp.float32),
        ],
    )(xp, W1, b1r, d0, d1)


def _tc_layer2_body(a_ref, z1_ref, dinv_ref, w_ref, b_ref,
                    y0_ref, y1_ref, z2_ref):
    dinv = dinv_ref[...]
    h = jnp.maximum(dinv * (a_ref[0] + a_ref[1]) + z1_ref[...], 0.0)
    xw = jnp.dot(h, w_ref[...], preferred_element_type=jnp.float32)
    y = xw * dinv
    y0_ref[...] = y[:, :HALF]
    y1_ref[...] = y[:, HALF:]
    z2_ref[...] = y * dinv + b_ref[...]


def _tc_layer2(aggp, z1, dinv, W2, b2r):
    return pl.pallas_call(
        _tc_layer2_body,
        grid=(_GRID1,),
        in_specs=[
            pl.BlockSpec((NC, _BLK, HIDDEN), lambda i: (0, i, 0)),
            pl.BlockSpec((_BLK, HIDDEN), lambda i: (i, 0)),
            pl.BlockSpec((_BLK, 1), lambda i: (i, 0)),
            pl.BlockSpec((HIDDEN, HIDDEN), lambda i: (0, 0)),
            pl.BlockSpec((1, HIDDEN), lambda i: (0, 0)),
        ],
        out_specs=[
            pl.BlockSpec((_BLK, HALF), lambda i: (i, 0)),
            pl.BlockSpec((_BLK, HALF), lambda i: (i, 0)),
            pl.BlockSpec((_BLK, HIDDEN), lambda i: (i, 0)),
        ],
        out_shape=[
            jax.ShapeDtypeStruct((NP, HALF), jnp.float32),
            jax.ShapeDtypeStruct((NP, HALF), jnp.float32),
            jax.ShapeDtypeStruct((NP, HIDDEN), jnp.float32),
        ],
    )(aggp, z1, dinv, W2, b2r)


_FBLK = 400
_GRIDF = N_NODES // _FBLK


def _tc_final_body(a_ref, z2_ref, dinv_ref, o_ref):
    o_ref[...] = dinv_ref[...] * (a_ref[0] + a_ref[1]) + z2_ref[...]


def _tc_final(aggp, z2, dinv):
    return pl.pallas_call(
        _tc_final_body,
        grid=(_GRIDF,),
        in_specs=[
            pl.BlockSpec((NC, _FBLK, HIDDEN), lambda i: (0, i, 0)),
            pl.BlockSpec((_FBLK, HIDDEN), lambda i: (i, 0)),
            pl.BlockSpec((_FBLK, 1), lambda i: (i, 0)),
        ],
        out_specs=pl.BlockSpec((_FBLK, HIDDEN), lambda i: (i, 0)),
        out_shape=jax.ShapeDtypeStruct((N_NODES, HIDDEN), jnp.float32),
    )(aggp, z2, dinv)


def kernel(x, edge_index, W1, b1, W2, b2):
    src = edge_index[0].astype(jnp.int32)
    dst = edge_index[1].astype(jnp.int32)
    e = src.shape[0]
    pad = jnp.full((EPAD - e,), N_NODES, jnp.int32)
    srcp = jnp.concatenate([src, pad])
    dstp = jnp.concatenate([dst, pad])
    src4 = srcp.reshape(NUNITS, KJS, GCHUNK)
    dst4 = dstp.reshape(NUNITS, KJS, GCHUNK)
    dst3 = dstp.reshape(NW, KJ, CHUNK)

    xp = jnp.pad(x, ((0, NP - x.shape[0]), (0, 0)))
    zeros128 = jnp.zeros((NP, HALF), jnp.float32)
    ones128 = jnp.ones((CHUNK, HALF), jnp.float32)
    b1r = b1.reshape(1, HIDDEN)
    b2r = b2.reshape(1, HIDDEN)

    degp = _sc_deg(dst3, zeros128, ones128)
    d0 = degp[0, :, :1]
    d1 = degp[1, :, :1]

    y0, y1, z1, dinv = _tc_layer1(xp, W1, b1r, d0, d1)
    aggp1 = _sc_agg(y0, y1, src4, dst4, zeros128)
    y20, y21, z2 = _tc_layer2(aggp1, z1, dinv, W2, b2r)
    aggp2 = _sc_agg(y20, y21, src4, dst4, zeros128)
    return _tc_final(aggp2, z2, dinv)


# speedup vs baseline: 1.2195x; 1.2195x over previous
import functools

import jax
import jax.numpy as jnp
from jax import lax
from jax.experimental import pallas as pl
from jax.experimental.pallas import tpu as pltpu
from jax.experimental.pallas import tpu_sc as plsc

N_NODES = 10000
N_EDGES = 320000
IN_DIM = 128
HIDDEN = 256

NP = 10240
NC = 2
NS = 16
NW = NC * NS
CHUNK = 128
KJ = 80
EPT = KJ * CHUNK
EPAD = NW * EPT
GCHUNK = 32
NBUF = 8
KJS = 32
UEDGES = KJS * GCHUNK
NUNITS = EPAD // UEDGES
MF = 15
MS = NUNITS // NS - MF
FAST_CORE = 0
ROWS_T = NP // NS
HALF = 128

_MESH = plsc.VectorSubcoreMesh(core_axis_name="c", subcore_axis_name="s",
                               num_cores=NC, num_subcores=NS)


def _sc_deg_body(dst3, zeros128, ones128, degp, dacc, didx, ones_v):
    c = lax.axis_index("c")
    s = lax.axis_index("s")
    wid = c * NS + s
    pltpu.sync_copy(zeros128.at[pl.ds(s * ROWS_T, ROWS_T)],
                    dacc.at[pl.ds(s * ROWS_T, ROWS_T)])
    pltpu.sync_copy(ones128, ones_v)
    pltpu.sync_copy(dst3.at[wid], didx)
    plsc.subcore_barrier()

    def body(j, carry):
        pltpu.sync_copy(ones_v, dacc.at[didx.at[j]], add=True)
        return carry

    lax.fori_loop(0, KJ, body, 0)
    plsc.subcore_barrier()
    pltpu.sync_copy(dacc.at[pl.ds(s * ROWS_T, ROWS_T)],
                    degp.at[c, pl.ds(s * ROWS_T, ROWS_T)])


def _sc_agg_body(y0, y1, src4, dst4, zeros128, aggp, acc, sidx, didx, bufs,
                 sems):
    c = lax.axis_index("c")
    s = lax.axis_index("s")
    base_u = jnp.where(c == FAST_CORE, s * MF, NS * MF + s * MS)

    for half in range(2):
        ysrc = y0 if half == 0 else y1
        pltpu.sync_copy(zeros128.at[pl.ds(s * ROWS_T, ROWS_T)],
                        acc.at[pl.ds(s * ROWS_T, ROWS_T)])
        plsc.subcore_barrier()

        def step(j, b, issue):
            pltpu.make_async_copy(ysrc.at[sidx.at[j]], bufs.at[b],
                                  sems.at[b]).wait()
            if issue:
                nb = (b + NBUF - 1) % NBUF
                pltpu.async_copy(ysrc.at[sidx.at[j + NBUF - 1]],
                                 bufs.at[nb], sems.at[nb])
            pltpu.sync_copy(bufs.at[b], acc.at[didx.at[j]], add=True)

        def do_unit(u):
            pltpu.sync_copy(src4.at[u], sidx)
            pltpu.sync_copy(dst4.at[u], didx)

            for b in range(NBUF - 1):
                pltpu.async_copy(ysrc.at[sidx.at[b]], bufs.at[b], sems.at[b])

            def body(i, carry):
                for b in range(NBUF):
                    step(NBUF * i + b, b, True)
                return carry

            lax.fori_loop(0, KJS // NBUF - 1, body, 0)
            base = KJS - NBUF
            step(base, base % NBUF, True)
            for j in range(base + 1, KJS):
                step(j, j % NBUF, False)

        for k in range(MF):
            if k < MS:
                do_unit(base_u + k)
            else:
                pl.when(c == FAST_CORE)(lambda k=k: do_unit(base_u + k))

        plsc.subcore_barrier()
        pltpu.sync_copy(
            acc.at[pl.ds(s * ROWS_T, ROWS_T)],
            aggp.at[c, pl.ds(s * ROWS_T, ROWS_T), pl.ds(half * HALF, HALF)])


_DEG_SCRATCH = [
    pltpu.VMEM_SHARED((NP, HALF), jnp.float32),
    pltpu.VMEM((KJ, CHUNK), jnp.int32),
    pltpu.VMEM((CHUNK, HALF), jnp.float32),
]
_AGG_SCRATCH = [
    pltpu.VMEM_SHARED((NP, HALF), jnp.float32),
    pltpu.VMEM((KJS, GCHUNK), jnp.int32),
    pltpu.VMEM((KJS, GCHUNK), jnp.int32),
    pltpu.VMEM((NBUF, GCHUNK, HALF), jnp.float32),
    pltpu.SemaphoreType.DMA((NBUF,)),
]
assert MF >= MS and NS * (MF + MS) == NUNITS

_sc_deg = pl.kernel(
    _sc_deg_body,
    out_type=jax.ShapeDtypeStruct((NC, NP, HALF), jnp.float32),
    mesh=_MESH,
    scratch_types=_DEG_SCRATCH,
)

_sc_agg = pl.kernel(
    _sc_agg_body,
    out_type=jax.ShapeDtypeStruct((NC, NP, HIDDEN), jnp.float32),
    mesh=_MESH,
    scratch_types=_AGG_SCRATCH,
)


_BLK = 512
_GRID1 = NP // _BLK


def _tc_layer1_body(x_ref, w_ref, b_ref, d0_ref, d1_ref,
                    y0_ref, y1_ref, z_ref, dinv_ref):
    xw = jnp.dot(x_ref[...], w_ref[...], preferred_element_type=jnp.float32)
    dinv = lax.rsqrt(d0_ref[...] + d1_ref[...] + 1.0)
    y = xw * dinv
    y0_ref[...] = y[:, :HALF]
    y1_ref[...] = y[:, HALF:]
    z_ref[...] = y * dinv + b_ref[...]
    dinv_ref[...] = dinv


def _tc_layer1(xp, W1, b1r, d0, d1):
    return pl.pallas_call(
        _tc_layer1_body,
        grid=(_GRID1,),
        in_specs=[
            pl.BlockSpec((_BLK, IN_DIM), lambda i: (i, 0)),
            pl.BlockSpec((IN_DIM, HIDDEN), lambda i: (0, 0)),
            pl.BlockSpec((1, HIDDEN), lambda i: (0, 0)),
            pl.BlockSpec((_BLK, 1), lambda i: (i, 0)),
            pl.BlockSpec((_BLK, 1), lambda i: (i, 0)),
        ],
        out_specs=[
            pl.BlockSpec((_BLK, HALF), lambda i: (i, 0)),
            pl.BlockSpec((_BLK, HALF), lambda i: (i, 0)),
            pl.BlockSpec((_BLK, HIDDEN), lambda i: (i, 0)),
            pl.BlockSpec((_BLK, 1), lambda i: (i, 0)),
        ],
        out_shape=[
            jax.ShapeDtypeStruct((NP, HALF), jnp.float32),
            jax.ShapeDtypeStruct((NP, HALF), jnp.float32),
            jax.ShapeDtypeStruct((NP, HIDDEN), jnp.float32),
            jax.ShapeDtypeStruct((NP, 1), jnp.float32),
        ],
    )(xp, W1, b1r, d0, d1)


def _tc_layer2_body(a_ref, z1_ref, dinv_ref, w_ref, b_ref,
                    y0_ref, y1_ref, z2_ref):
    dinv = dinv_ref[...]
    h = jnp.maximum(dinv * (a_ref[0] + a_ref[1]) + z1_ref[...], 0.0)
    xw = jnp.dot(h, w_ref[...], preferred_element_type=jnp.float32)
    y = xw * dinv
    y0_ref[...] = y[:, :HALF]
    y1_ref[...] = y[:, HALF:]
    z2_ref[...] = y * dinv + b_ref[...]


def _tc_layer2(aggp, z1, dinv, W2, b2r):
    return pl.pallas_call(
        _tc_layer2_body,
        grid=(_GRID1,),
        in_specs=[
            pl.BlockSpec((NC, _BLK, HIDDEN), lambda i: (0, i, 0)),
            pl.BlockSpec((_BLK, HIDDEN), lambda i: (i, 0)),
            pl.BlockSpec((_BLK, 1), lambda i: (i, 0)),
            pl.BlockSpec((HIDDEN, HIDDEN), lambda i: (0, 0)),
            pl.BlockSpec((1, HIDDEN), lambda i: (0, 0)),
        ],
        out_specs=[
            pl.BlockSpec((_BLK, HALF), lambda i: (i, 0)),
            pl.BlockSpec((_BLK, HALF), lambda i: (i, 0)),
            pl.BlockSpec((_BLK, HIDDEN), lambda i: (i, 0)),
        ],
        out_shape=[
            jax.ShapeDtypeStruct((NP, HALF), jnp.float32),
            jax.ShapeDtypeStruct((NP, HALF), jnp.float32),
            jax.ShapeDtypeStruct((NP, HIDDEN), jnp.float32),
        ],
    )(aggp, z1, dinv, W2, b2r)


_FBLK = 400
_GRIDF = N_NODES // _FBLK


def _tc_final_body(a_ref, z2_ref, dinv_ref, o_ref):
    o_ref[...] = dinv_ref[...] * (a_ref[0] + a_ref[1]) + z2_ref[...]


def _tc_final(aggp, z2, dinv):
    return pl.pallas_call(
        _tc_final_body,
        grid=(_GRIDF,),
        in_specs=[
            pl.BlockSpec((NC, _FBLK, HIDDEN), lambda i: (0, i, 0)),
            pl.BlockSpec((_FBLK, HIDDEN), lambda i: (i, 0)),
            pl.BlockSpec((_FBLK, 1), lambda i: (i, 0)),
        ],
        out_specs=pl.BlockSpec((_FBLK, HIDDEN), lambda i: (i, 0)),
        out_shape=jax.ShapeDtypeStruct((N_NODES, HIDDEN), jnp.float32),
    )(aggp, z2, dinv)


def kernel(x, edge_index, W1, b1, W2, b2):
    src = edge_index[0].astype(jnp.int32)
    dst = edge_index[1].astype(jnp.int32)
    e = src.shape[0]
    pad = jnp.full((EPAD - e,), N_NODES, jnp.int32)
    srcp = jnp.concatenate([src, pad])
    dstp = jnp.concatenate([dst, pad])
    src4 = srcp.reshape(NUNITS, KJS, GCHUNK)
    dst4 = dstp.reshape(NUNITS, KJS, GCHUNK)
    dst3 = dstp.reshape(NW, KJ, CHUNK)

    xp = jnp.pad(x, ((0, NP - x.shape[0]), (0, 0)))
    zeros128 = jnp.zeros((NP, HALF), jnp.float32)
    ones128 = jnp.ones((CHUNK, HALF), jnp.float32)
    b1r = b1.reshape(1, HIDDEN)
    b2r = b2.reshape(1, HIDDEN)

    degp = _sc_deg(dst3, zeros128, ones128)
    d0 = degp[0, :, :1]
    d1 = degp[1, :, :1]

    y0, y1, z1, dinv = _tc_layer1(xp, W1, b1r, d0, d1)
    aggp1 = _sc_agg(y0, y1, src4, dst4, zeros128)
    y20, y21, z2 = _tc_layer2(aggp1, z1, dinv, W2, b2r)
    aggp2 = _sc_agg(y20, y21, src4, dst4, zeros128)
    return _tc_final(aggp2, z2, dinv)


# speedup vs baseline: 1.2293x; 1.0080x over previous
import functools

import jax
import jax.numpy as jnp
from jax import lax
from jax.experimental import pallas as pl
from jax.experimental.pallas import tpu as pltpu
from jax.experimental.pallas import tpu_sc as plsc

N_NODES = 10000
N_EDGES = 320000
IN_DIM = 128
HIDDEN = 256

NP = 10240
NC = 2
NS = 16
NW = NC * NS
CHUNK = 128
KJ = 80
EPT = KJ * CHUNK
EPAD = NW * EPT
GCHUNK = 64
NBUF = 4
KJS = 16
UEDGES = KJS * GCHUNK
NUNITS = EPAD // UEDGES
MF = 15
MS = NUNITS // NS - MF
FAST_CORE = 0
ROWS_T = NP // NS
HALF = 128

_MESH = plsc.VectorSubcoreMesh(core_axis_name="c", subcore_axis_name="s",
                               num_cores=NC, num_subcores=NS)


def _sc_deg_body(dst3, zeros128, ones128, degp, dacc, didx, ones_v):
    c = lax.axis_index("c")
    s = lax.axis_index("s")
    wid = c * NS + s
    pltpu.sync_copy(zeros128.at[pl.ds(s * ROWS_T, ROWS_T)],
                    dacc.at[pl.ds(s * ROWS_T, ROWS_T)])
    pltpu.sync_copy(ones128, ones_v)
    pltpu.sync_copy(dst3.at[wid], didx)
    plsc.subcore_barrier()

    def body(j, carry):
        pltpu.sync_copy(ones_v, dacc.at[didx.at[j]], add=True)
        return carry

    lax.fori_loop(0, KJ, body, 0)
    plsc.subcore_barrier()
    pltpu.sync_copy(dacc.at[pl.ds(s * ROWS_T, ROWS_T)],
                    degp.at[c, pl.ds(s * ROWS_T, ROWS_T)])


def _sc_agg_body(y0, y1, src4, dst4, zeros128, aggp, acc, sidx, didx, bufs,
                 sems):
    c = lax.axis_index("c")
    s = lax.axis_index("s")
    base_u = jnp.where(c == FAST_CORE, s * MF, NS * MF + s * MS)

    for half in range(2):
        ysrc = y0 if half == 0 else y1
        pltpu.sync_copy(zeros128.at[pl.ds(s * ROWS_T, ROWS_T)],
                        acc.at[pl.ds(s * ROWS_T, ROWS_T)])
        plsc.subcore_barrier()

        def step(j, b, issue):
            pltpu.make_async_copy(ysrc.at[sidx.at[j]], bufs.at[b],
                                  sems.at[b]).wait()
            if issue:
                nb = (b + NBUF - 1) % NBUF
                pltpu.async_copy(ysrc.at[sidx.at[j + NBUF - 1]],
                                 bufs.at[nb], sems.at[nb])
            pltpu.sync_copy(bufs.at[b], acc.at[didx.at[j]], add=True)

        def do_unit(u):
            pltpu.sync_copy(src4.at[u], sidx)
            pltpu.sync_copy(dst4.at[u], didx)

            for b in range(NBUF - 1):
                pltpu.async_copy(ysrc.at[sidx.at[b]], bufs.at[b], sems.at[b])

            def body(i, carry):
                for b in range(NBUF):
                    step(NBUF * i + b, b, True)
                return carry

            lax.fori_loop(0, KJS // NBUF - 1, body, 0)
            base = KJS - NBUF
            step(base, base % NBUF, True)
            for j in range(base + 1, KJS):
                step(j, j % NBUF, False)

        for k in range(MF):
            if k < MS:
                do_unit(base_u + k)
            else:
                pl.when(c == FAST_CORE)(lambda k=k: do_unit(base_u + k))

        plsc.subcore_barrier()
        pltpu.sync_copy(
            acc.at[pl.ds(s * ROWS_T, ROWS_T)],
            aggp.at[c, pl.ds(s * ROWS_T, ROWS_T), pl.ds(half * HALF, HALF)])


_DEG_SCRATCH = [
    pltpu.VMEM_SHARED((NP, HALF), jnp.float32),
    pltpu.VMEM((KJ, CHUNK), jnp.int32),
    pltpu.VMEM((CHUNK, HALF), jnp.float32),
]
_AGG_SCRATCH = [
    pltpu.VMEM_SHARED((NP, HALF), jnp.float32),
    pltpu.VMEM((KJS, GCHUNK), jnp.int32),
    pltpu.VMEM((KJS, GCHUNK), jnp.int32),
    pltpu.VMEM((NBUF, GCHUNK, HALF), jnp.float32),
    pltpu.SemaphoreType.DMA((NBUF,)),
]
assert MF >= MS and NS * (MF + MS) == NUNITS

_sc_deg = pl.kernel(
    _sc_deg_body,
    out_type=jax.ShapeDtypeStruct((NC, NP, HALF), jnp.float32),
    mesh=_MESH,
    scratch_types=_DEG_SCRATCH,
)

_sc_agg = pl.kernel(
    _sc_agg_body,
    out_type=jax.ShapeDtypeStruct((NC, NP, HIDDEN), jnp.float32),
    mesh=_MESH,
    scratch_types=_AGG_SCRATCH,
)


_BLK = 512
_GRID1 = NP // _BLK


def _tc_layer1_body(x_ref, w_ref, b_ref, d0_ref, d1_ref,
                    y0_ref, y1_ref, z_ref, dinv_ref):
    xw = jnp.dot(x_ref[...], w_ref[...], preferred_element_type=jnp.float32)
    dinv = lax.rsqrt(d0_ref[...] + d1_ref[...] + 1.0)
    y = xw * dinv
    y0_ref[...] = y[:, :HALF]
    y1_ref[...] = y[:, HALF:]
    z_ref[...] = y * dinv + b_ref[...]
    dinv_ref[...] = dinv


def _tc_layer1(xp, W1, b1r, d0, d1):
    return pl.pallas_call(
        _tc_layer1_body,
        grid=(_GRID1,),
        in_specs=[
            pl.BlockSpec((_BLK, IN_DIM), lambda i: (i, 0)),
            pl.BlockSpec((IN_DIM, HIDDEN), lambda i: (0, 0)),
            pl.BlockSpec((1, HIDDEN), lambda i: (0, 0)),
            pl.BlockSpec((_BLK, 1), lambda i: (i, 0)),
            pl.BlockSpec((_BLK, 1), lambda i: (i, 0)),
        ],
        out_specs=[
            pl.BlockSpec((_BLK, HALF), lambda i: (i, 0)),
            pl.BlockSpec((_BLK, HALF), lambda i: (i, 0)),
            pl.BlockSpec((_BLK, HIDDEN), lambda i: (i, 0)),
            pl.BlockSpec((_BLK, 1), lambda i: (i, 0)),
        ],
        out_shape=[
            jax.ShapeDtypeStruct((NP, HALF), jnp.float32),
            jax.ShapeDtypeStruct((NP, HALF), jnp.float32),
            jax.ShapeDtypeStruct((NP, HIDDEN), jnp.float32),
            jax.ShapeDtypeStruct((NP, 1), jnp.float32),
        ],
    )(xp, W1, b1r, d0, d1)


def _tc_layer2_body(a_ref, z1_ref, dinv_ref, w_ref, b_ref,
                    y0_ref, y1_ref, z2_ref):
    dinv = dinv_ref[...]
    h = jnp.maximum(dinv * (a_ref[0] + a_ref[1]) + z1_ref[...], 0.0)
    xw = jnp.dot(h, w_ref[...], preferred_element_type=jnp.float32)
    y = xw * dinv
    y0_ref[...] = y[:, :HALF]
    y1_ref[...] = y[:, HALF:]
    z2_ref[...] = y * dinv + b_ref[...]


def _tc_layer2(aggp, z1, dinv, W2, b2r):
    return pl.pallas_call(
        _tc_layer2_body,
        grid=(_GRID1,),
        in_specs=[
            pl.BlockSpec((NC, _BLK, HIDDEN), lambda i: (0, i, 0)),
            pl.BlockSpec((_BLK, HIDDEN), lambda i: (i, 0)),
            pl.BlockSpec((_BLK, 1), lambda i: (i, 0)),
            pl.BlockSpec((HIDDEN, HIDDEN), lambda i: (0, 0)),
            pl.BlockSpec((1, HIDDEN), lambda i: (0, 0)),
        ],
        out_specs=[
            pl.BlockSpec((_BLK, HALF), lambda i: (i, 0)),
            pl.BlockSpec((_BLK, HALF), lambda i: (i, 0)),
            pl.BlockSpec((_BLK, HIDDEN), lambda i: (i, 0)),
        ],
        out_shape=[
            jax.ShapeDtypeStruct((NP, HALF), jnp.float32),
            jax.ShapeDtypeStruct((NP, HALF), jnp.float32),
            jax.ShapeDtypeStruct((NP, HIDDEN), jnp.float32),
        ],
    )(aggp, z1, dinv, W2, b2r)


_FBLK = 400
_GRIDF = N_NODES // _FBLK


def _tc_final_body(a_ref, z2_ref, dinv_ref, o_ref):
    o_ref[...] = dinv_ref[...] * (a_ref[0] + a_ref[1]) + z2_ref[...]


def _tc_final(aggp, z2, dinv):
    return pl.pallas_call(
        _tc_final_body,
        grid=(_GRIDF,),
        in_specs=[
            pl.BlockSpec((NC, _FBLK, HIDDEN), lambda i: (0, i, 0)),
            pl.BlockSpec((_FBLK, HIDDEN), lambda i: (i, 0)),
            pl.BlockSpec((_FBLK, 1), lambda i: (i, 0)),
        ],
        out_specs=pl.BlockSpec((_FBLK, HIDDEN), lambda i: (i, 0)),
        out_shape=jax.ShapeDtypeStruct((N_NODES, HIDDEN), jnp.float32),
    )(aggp, z2, dinv)


def kernel(x, edge_index, W1, b1, W2, b2):
    src = edge_index[0].astype(jnp.int32)
    dst = edge_index[1].astype(jnp.int32)
    e = src.shape[0]
    pad = jnp.full((EPAD - e,), N_NODES, jnp.int32)
    srcp = jnp.concatenate([src, pad])
    dstp = jnp.concatenate([dst, pad])
    src4 = srcp.reshape(NUNITS, KJS, GCHUNK)
    dst4 = dstp.reshape(NUNITS, KJS, GCHUNK)
    dst3 = dstp.reshape(NW, KJ, CHUNK)

    xp = jnp.pad(x, ((0, NP - x.shape[0]), (0, 0)))
    zeros128 = jnp.zeros((NP, HALF), jnp.float32)
    ones128 = jnp.ones((CHUNK, HALF), jnp.float32)
    b1r = b1.reshape(1, HIDDEN)
    b2r = b2.reshape(1, HIDDEN)

    degp = _sc_deg(dst3, zeros128, ones128)
    d0 = degp[0, :, :1]
    d1 = degp[1, :, :1]

    y0, y1, z1, dinv = _tc_layer1(xp, W1, b1r, d0, d1)
    aggp1 = _sc_agg(y0, y1, src4, dst4, zeros128)
    y20, y21, z2 = _tc_layer2(aggp1, z1, dinv, W2, b2r)
    aggp2 = _sc_agg(y20, y21, src4, dst4, zeros128)
    return _tc_final(aggp2, z2, dinv)
